# baseline (speedup 1.0000x reference)
import functools

import jax
import jax.numpy as jnp
from jax import lax
from jax.experimental import pallas as pl
from jax.experimental.pallas import tpu as pltpu
from jax.experimental.pallas import tpu_sc as plsc

SIZE = 128
VEC = SIZE * (SIZE + 1) // 2
L = 16
NCHUNK = SIZE // L


def _softplus16(x):
    y = jnp.exp(-jnp.abs(x))
    w = y / (2.0 + y)
    w2 = w * w
    p = 1.0 / 7.0 + w2 * (1.0 / 9.0)
    p = 1.0 / 5.0 + w2 * p
    p = 1.0 / 3.0 + w2 * p
    log1p = 2.0 * w * (1.0 + w2 * p)
    return jnp.maximum(x, 0.0) + log1p


def _sc_body(per_worker, v_hbm, out_hbm, v_buf0, v_buf1, o_buf0, o_buf1,
             in_sem0, in_sem1, out_sem0, out_sem1):
    info = plsc.get_sparse_core_info()
    nc = info.num_cores
    wid = lax.axis_index("s") * nc + lax.axis_index("c")
    base = wid * per_worker
    iota = lax.iota(jnp.int32, L)
    v_bufs = (v_buf0, v_buf1)
    o_bufs = (o_buf0, o_buf1)
    in_sems = (in_sem0, in_sem1)
    out_sems = (out_sem0, out_sem1)

    def in_copy(t, p):
        return pltpu.make_async_copy(v_hbm.at[base + t], v_bufs[p], in_sems[p])

    def out_copy(t, p):
        return pltpu.make_async_copy(o_bufs[p], out_hbm.at[base + t],
                                     out_sems[p])

    zero = jnp.zeros((L,), jnp.float32)

    def zrow(i, _):
        for p in range(2):
            for jv in range(NCHUNK):
                o_bufs[p][i, pl.ds(jv * L, L)] = zero
        return 0

    lax.fori_loop(0, SIZE, zrow, 0)

    def compute(p):
        vb = v_bufs[p]
        ob = o_bufs[p]

        for jv in range(NCHUNK - 1):
            c0v = iota + jv * L

            @plsc.parallel_loop(L * (jv + 1), SIZE, unroll=4)
            def _full(i):
                tri = (i * (i + 1)) >> 1
                seg = plsc.load_gather(vb, [tri + c0v])
                ob[i, pl.ds(jv * L, L)] = seg

        for jv in range(NCHUNK):
            c0v = iota + jv * L

            @plsc.parallel_loop(0, L, unroll=2)
            def _band(r):
                i = jv * L + r
                tri = (i * (i + 1)) >> 1
                seg = plsc.load_gather(vb, [tri + c0v])
                ob[i, pl.ds(jv * L, L)] = jnp.where(iota < r, seg, 0.0)

        for c in range(NCHUNK):
            ivec = iota + c * L
            src = (ivec * (ivec + 3)) >> 1
            x = plsc.load_gather(vb, [src])
            plsc.store_scatter(ob, [ivec, ivec], _softplus16(x))

    in_copy(0, 0).start()
    in_copy(1, 1).start()

    def outer(tt, _):
        for p in range(2):
            t = tt * 2 + p
            in_copy(t, p).wait()

            @pl.when(t >= 2)
            def _wait_out():
                out_copy(t - 2, p).wait()

            compute(p)
            out_copy(t, p).start()

            @pl.when(t + 2 < per_worker)
            def _next_in():
                in_copy(t + 2, p).start()
        return 0

    lax.fori_loop(0, per_worker // 2, outer, 0)
    out_copy(per_worker - 2, 0).wait()
    out_copy(per_worker - 1, 1).wait()


def kernel(L_vec):
    batch = L_vec.shape[0]
    info = plsc.get_sparse_core_info()
    n_workers = info.num_cores * info.num_subcores
    per_worker = batch // n_workers
    mesh = plsc.VectorSubcoreMesh(core_axis_name="c", subcore_axis_name="s")
    k = functools.partial(
        pl.kernel,
        out_type=jax.ShapeDtypeStruct((batch, SIZE, SIZE), jnp.float32),
        mesh=mesh,
        compiler_params=pltpu.CompilerParams(
            needs_layout_passes=False, use_tc_tiling_on_sc=True),
        scratch_types=[
            pltpu.VMEM((VEC,), jnp.float32),
            pltpu.VMEM((VEC,), jnp.float32),
            pltpu.VMEM((SIZE, SIZE), jnp.float32),
            pltpu.VMEM((SIZE, SIZE), jnp.float32),
            pltpu.SemaphoreType.DMA,
            pltpu.SemaphoreType.DMA,
            pltpu.SemaphoreType.DMA,
            pltpu.SemaphoreType.DMA,
        ],
    )(functools.partial(_sc_body, per_worker))
    return k(L_vec)


# speedup vs baseline: 1.0293x; 1.0293x over previous
import functools

import jax
import jax.numpy as jnp
from jax import lax
from jax.experimental import pallas as pl
from jax.experimental.pallas import tpu as pltpu
from jax.experimental.pallas import tpu_sc as plsc

SIZE = 128
VEC = SIZE * (SIZE + 1) // 2
L = 16
NCHUNK = SIZE // L


def _softplus16(x):
    y = jnp.exp(-jnp.abs(x))
    w = y / (2.0 + y)
    w2 = w * w
    p = 1.0 / 7.0 + w2 * (1.0 / 9.0)
    p = 1.0 / 5.0 + w2 * p
    p = 1.0 / 3.0 + w2 * p
    log1p = 2.0 * w * (1.0 + w2 * p)
    return jnp.maximum(x, 0.0) + log1p


def _sc_body(per_worker, v_hbm, out_hbm, v_buf0, v_buf1, o_buf0, o_buf1,
             in_sem0, in_sem1, out_sem0, out_sem1):
    info = plsc.get_sparse_core_info()
    nc = info.num_cores
    wid = lax.axis_index("s") * nc + lax.axis_index("c")
    base = wid * per_worker
    iota = lax.iota(jnp.int32, L)
    v_bufs = (v_buf0, v_buf1)
    o_bufs = (o_buf0, o_buf1)
    in_sems = (in_sem0, in_sem1)
    out_sems = (out_sem0, out_sem1)

    def in_copy(t, p):
        return pltpu.make_async_copy(v_hbm.at[base + t], v_bufs[p], in_sems[p])

    def out_copy(t, p):
        return pltpu.make_async_copy(o_bufs[p], out_hbm.at[base + t],
                                     out_sems[p])

    zero = jnp.zeros((L,), jnp.float32)

    def zrow(i, _):
        for p in range(2):
            for jv in range(NCHUNK):
                o_bufs[p][i, pl.ds(jv * L, L)] = zero
        return 0

    lax.fori_loop(0, SIZE, zrow, 0)

    def compute(p):
        vb = v_bufs[p]
        ob = o_bufs[p]

        for jv in range(NCHUNK - 1):
            c0v = iota + jv * L

            @plsc.parallel_loop(L * (jv + 1), SIZE, unroll=8)
            def _full(i):
                tri = (i * (i + 1)) >> 1
                seg = plsc.load_gather(vb, [tri + c0v])
                ob[i, pl.ds(jv * L, L)] = seg

        for jv in range(NCHUNK):
            c0v = iota + jv * L

            @plsc.parallel_loop(0, L, unroll=4)
            def _band(r):
                i = jv * L + r
                tri = (i * (i + 1)) >> 1
                seg = plsc.load_gather(vb, [tri + c0v])
                ob[i, pl.ds(jv * L, L)] = jnp.where(iota < r, seg, 0.0)

        for c in range(NCHUNK):
            ivec = iota + c * L
            src = (ivec * (ivec + 3)) >> 1
            x = plsc.load_gather(vb, [src])
            plsc.store_scatter(ob, [ivec, ivec], _softplus16(x))

    in_copy(0, 0).start()
    in_copy(1, 1).start()

    def outer(tt, _):
        for p in range(2):
            t = tt * 2 + p
            in_copy(t, p).wait()

            @pl.when(t >= 2)
            def _wait_out():
                out_copy(t - 2, p).wait()

            compute(p)
            out_copy(t, p).start()

            @pl.when(t + 2 < per_worker)
            def _next_in():
                in_copy(t + 2, p).start()
        return 0

    lax.fori_loop(0, per_worker // 2, outer, 0)
    out_copy(per_worker - 2, 0).wait()
    out_copy(per_worker - 1, 1).wait()


def kernel(L_vec):
    batch = L_vec.shape[0]
    info = plsc.get_sparse_core_info()
    n_workers = info.num_cores * info.num_subcores
    per_worker = batch // n_workers
    mesh = plsc.VectorSubcoreMesh(core_axis_name="c", subcore_axis_name="s")
    k = functools.partial(
        pl.kernel,
        out_type=jax.ShapeDtypeStruct((batch, SIZE, SIZE), jnp.float32),
        mesh=mesh,
        compiler_params=pltpu.CompilerParams(
            needs_layout_passes=False, disable_bounds_checks=True),
        scratch_types=[
            pltpu.VMEM((VEC,), jnp.float32),
            pltpu.VMEM((VEC,), jnp.float32),
            pltpu.VMEM((SIZE, SIZE), jnp.float32),
            pltpu.VMEM((SIZE, SIZE), jnp.float32),
            pltpu.SemaphoreType.DMA,
            pltpu.SemaphoreType.DMA,
            pltpu.SemaphoreType.DMA,
            pltpu.SemaphoreType.DMA,
        ],
    )(functools.partial(_sc_body, per_worker))
    return k(L_vec)


# speedup vs baseline: 1.1679x; 1.1346x over previous
import functools

import jax
import jax.numpy as jnp
from jax import lax
from jax.experimental import pallas as pl
from jax.experimental.pallas import tpu as pltpu
from jax.experimental.pallas import tpu_sc as plsc

SIZE = 128
VEC = SIZE * (SIZE + 1) // 2
L = 16
NCHUNK = SIZE // L


def _softplus16(x):
    y = jnp.exp(-jnp.abs(x))
    w = y / (2.0 + y)
    w2 = w * w
    p = 1.0 / 7.0 + w2 * (1.0 / 9.0)
    p = 1.0 / 5.0 + w2 * p
    p = 1.0 / 3.0 + w2 * p
    log1p = 2.0 * w * (1.0 + w2 * p)
    return jnp.maximum(x, 0.0) + log1p


def _sc_body(per_worker, v_hbm, out_hbm, v_buf0, v_buf1, o_buf0, o_buf1,
             in_sem0, in_sem1, out_sem0, out_sem1):
    info = plsc.get_sparse_core_info()
    nc = info.num_cores
    wid = lax.axis_index("s") * nc + lax.axis_index("c")
    base = wid * per_worker
    iota = lax.iota(jnp.int32, L)
    v_bufs = (v_buf0, v_buf1)
    o_bufs = (o_buf0, o_buf1)
    in_sems = (in_sem0, in_sem1)
    out_sems = (out_sem0, out_sem1)

    def in_copy(t, p):
        return pltpu.make_async_copy(v_hbm.at[base + t], v_bufs[p], in_sems[p])

    def out_copy(t, p):
        return pltpu.make_async_copy(o_bufs[p], out_hbm.at[base + t],
                                     out_sems[p])

    zero = jnp.zeros((L,), jnp.float32)

    def zrow(i, _):
        for p in range(2):
            for jv in range(NCHUNK):
                o_bufs[p][i, pl.ds(jv * L, L)] = zero
        return 0

    lax.fori_loop(0, SIZE, zrow, 0)

    def compute(p):
        vb = v_bufs[p]
        ob = o_bufs[p]

        for jv in range(NCHUNK - 1):
            c0v = iota + jv * L

            @plsc.parallel_loop(L * (jv + 1), SIZE, unroll=8)
            def _full(i):
                tri = (i * (i + 1)) >> 1
                seg = plsc.load_gather(vb, [tri + c0v])
                ob[i, pl.ds(jv * L, L)] = seg

        @plsc.parallel_loop(0, SIZE, unroll=4)
        def _band(i):
            tri = (i * (i + 1)) >> 1
            c0 = pl.multiple_of((i >> 4) * L, L)
            seg = plsc.load_gather(vb, [tri + c0 + iota])
            ob[i, pl.ds(c0, L)] = jnp.where(iota < (i & (L - 1)), seg, 0.0)

        for c in range(NCHUNK):
            ivec = iota + c * L
            src = (ivec * (ivec + 3)) >> 1
            x = plsc.load_gather(vb, [src])
            plsc.store_scatter(ob, [ivec, ivec], _softplus16(x))

    in_copy(0, 0).start()
    in_copy(1, 1).start()

    def outer(tt, _):
        for p in range(2):
            t = tt * 2 + p
            in_copy(t, p).wait()

            @pl.when(t >= 2)
            def _wait_out():
                out_copy(t - 2, p).wait()

            compute(p)
            out_copy(t, p).start()

            @pl.when(t + 2 < per_worker)
            def _next_in():
                in_copy(t + 2, p).start()
        return 0

    lax.fori_loop(0, per_worker // 2, outer, 0)
    out_copy(per_worker - 2, 0).wait()
    out_copy(per_worker - 1, 1).wait()


def kernel(L_vec):
    batch = L_vec.shape[0]
    info = plsc.get_sparse_core_info()
    n_workers = info.num_cores * info.num_subcores
    per_worker = batch // n_workers
    mesh = plsc.VectorSubcoreMesh(core_axis_name="c", subcore_axis_name="s")
    k = functools.partial(
        pl.kernel,
        out_type=jax.ShapeDtypeStruct((batch, SIZE, SIZE), jnp.float32),
        mesh=mesh,
        compiler_params=pltpu.CompilerParams(
            needs_layout_passes=False, disable_bounds_checks=True),
        scratch_types=[
            pltpu.VMEM((VEC,), jnp.float32),
            pltpu.VMEM((VEC,), jnp.float32),
            pltpu.VMEM((SIZE, SIZE), jnp.float32),
            pltpu.VMEM((SIZE, SIZE), jnp.float32),
            pltpu.SemaphoreType.DMA,
            pltpu.SemaphoreType.DMA,
            pltpu.SemaphoreType.DMA,
            pltpu.SemaphoreType.DMA,
        ],
    )(functools.partial(_sc_body, per_worker))
    return k(L_vec)


# speedup vs baseline: 1.1901x; 1.0190x over previous
import functools

import jax
import jax.numpy as jnp
from jax import lax
from jax.experimental import pallas as pl
from jax.experimental.pallas import tpu as pltpu
from jax.experimental.pallas import tpu_sc as plsc

SIZE = 128
VEC = SIZE * (SIZE + 1) // 2
L = 16
NCHUNK = SIZE // L


def _softplus16(x):
    y = jnp.exp(-jnp.abs(x))
    w = y / (2.0 + y)
    w2 = w * w
    p = 1.0 / 7.0 + w2 * (1.0 / 9.0)
    p = 1.0 / 5.0 + w2 * p
    p = 1.0 / 3.0 + w2 * p
    log1p = 2.0 * w * (1.0 + w2 * p)
    return jnp.maximum(x, 0.0) + log1p


def _sc_body(per_worker, v_hbm, out_hbm, v_buf0, v_buf1, o_buf0, o_buf1,
             in_sem0, in_sem1, out_sem0, out_sem1):
    info = plsc.get_sparse_core_info()
    nc = info.num_cores
    wid = lax.axis_index("s") * nc + lax.axis_index("c")
    base = wid * per_worker
    iota = lax.iota(jnp.int32, L)
    v_bufs = (v_buf0, v_buf1)
    o_bufs = (o_buf0, o_buf1)
    in_sems = (in_sem0, in_sem1)
    out_sems = (out_sem0, out_sem1)

    def in_copy(t, p):
        return pltpu.make_async_copy(v_hbm.at[base + t], v_bufs[p], in_sems[p])

    def out_copy(t, p):
        return pltpu.make_async_copy(o_bufs[p], out_hbm.at[base + t],
                                     out_sems[p])

    zero = jnp.zeros((L,), jnp.float32)

    def zrow(i, _):
        for p in range(2):
            for jv in range(NCHUNK):
                o_bufs[p][i, pl.ds(jv * L, L)] = zero
        return 0

    lax.fori_loop(0, SIZE, zrow, 0)

    def compute(p):
        vb = v_bufs[p]
        ob = o_bufs[p]

        for jv in range(NCHUNK - 1):
            c0v = iota + jv * L

            @plsc.parallel_loop(L * (jv + 1), SIZE, unroll=16)
            def _full(i):
                tri = (i * (i + 1)) >> 1
                seg = plsc.load_gather(vb, [tri + c0v])
                ob[i, pl.ds(jv * L, L)] = seg

        @plsc.parallel_loop(0, SIZE, unroll=8)
        def _band(i):
            tri = (i * (i + 1)) >> 1
            c0 = pl.multiple_of((i >> 4) * L, L)
            seg = plsc.load_gather(vb, [tri + c0 + iota])
            ob[i, pl.ds(c0, L)] = jnp.where(iota < (i & (L - 1)), seg, 0.0)

        for c in range(NCHUNK):
            ivec = iota + c * L
            src = (ivec * (ivec + 3)) >> 1
            x = plsc.load_gather(vb, [src])
            plsc.store_scatter(ob, [ivec, ivec], _softplus16(x))

    in_copy(0, 0).start()
    in_copy(1, 1).start()

    def outer(tt, _):
        for p in range(2):
            t = tt * 2 + p
            in_copy(t, p).wait()

            @pl.when(t >= 2)
            def _wait_out():
                out_copy(t - 2, p).wait()

            compute(p)
            out_copy(t, p).start()

            @pl.when(t + 2 < per_worker)
            def _next_in():
                in_copy(t + 2, p).start()
        return 0

    lax.fori_loop(0, per_worker // 2, outer, 0)
    out_copy(per_worker - 2, 0).wait()
    out_copy(per_worker - 1, 1).wait()


def kernel(L_vec):
    batch = L_vec.shape[0]
    info = plsc.get_sparse_core_info()
    n_workers = info.num_cores * info.num_subcores
    per_worker = batch // n_workers
    mesh = plsc.VectorSubcoreMesh(core_axis_name="c", subcore_axis_name="s")
    k = functools.partial(
        pl.kernel,
        out_type=jax.ShapeDtypeStruct((batch, SIZE, SIZE), jnp.float32),
        mesh=mesh,
        compiler_params=pltpu.CompilerParams(
            needs_layout_passes=False, disable_bounds_checks=True),
        scratch_types=[
            pltpu.VMEM((VEC,), jnp.float32),
            pltpu.VMEM((VEC,), jnp.float32),
            pltpu.VMEM((SIZE, SIZE), jnp.float32),
            pltpu.VMEM((SIZE, SIZE), jnp.float32),
            pltpu.SemaphoreType.DMA,
            pltpu.SemaphoreType.DMA,
            pltpu.SemaphoreType.DMA,
            pltpu.SemaphoreType.DMA,
        ],
    )(functools.partial(_sc_body, per_worker))
    return k(L_vec)
